# Initial kernel scaffold; baseline (speedup 1.0000x reference)
import jax
import jax.numpy as jnp
from jax.experimental import pallas as pl


def kernel(U, vals0, rows0, cols0, vals1, rows1, cols1, vals2, rows2, cols2, bias):
    raise NotImplementedError("write your pallas kernel here")


# speedup vs baseline: 1.2683x; 1.2683x over previous
import functools

import jax
import jax.numpy as jnp
from jax.experimental import pallas as pl
from jax.experimental.pallas import tpu as pltpu

N = 4096
B = 256
KBLK = 512
NKB = N // KBLK


def _chain_body(u_ref, t_ref, bias_ref, out_ref, x_ref, acc_ref):
    i = pl.program_id(0)
    k = pl.program_id(1)

    @pl.when(jnp.logical_and(i == 0, k == 0))
    def _init_x():
        x_ref[...] = u_ref[...]

    @pl.when(k == 0)
    def _init_acc():
        acc_ref[...] = jnp.zeros_like(acc_ref)

    xblk = x_ref[:, pl.ds(k * KBLK, KBLK)]
    acc_ref[...] += jnp.dot(xblk, t_ref[0], preferred_element_type=jnp.float32)

    @pl.when(k == NKB - 1)
    def _finish_factor():
        x_ref[...] = acc_ref[...]

    @pl.when(jnp.logical_and(i == 2, k == NKB - 1))
    def _emit():
        out_ref[...] = acc_ref[...] + bias_ref[...]


@jax.jit
def _matmul_chain(U, T_stack, bias):
    return pl.pallas_call(
        _chain_body,
        grid=(3, NKB),
        in_specs=[
            pl.BlockSpec((B, N), lambda i, k: (0, 0)),
            pl.BlockSpec((1, KBLK, N), lambda i, k: (i, k, 0)),
            pl.BlockSpec((1, N), lambda i, k: (0, 0)),
        ],
        out_specs=pl.BlockSpec((B, N), lambda i, k: (0, 0)),
        out_shape=jax.ShapeDtypeStruct((B, N), jnp.float32),
        scratch_shapes=[
            pltpu.VMEM((B, N), jnp.float32),
            pltpu.VMEM((B, N), jnp.float32),
        ],
        compiler_params=pltpu.CompilerParams(
            dimension_semantics=("arbitrary", "arbitrary"),
        ),
    )(U, T_stack, bias.reshape(1, N))


def _densify_t(vals, rows, cols):
    flat = cols.astype(jnp.int32) * N + rows.astype(jnp.int32)
    d = jnp.zeros((N * N,), jnp.float32).at[flat].add(vals)
    return d.reshape(N, N)


def kernel(U, vals0, rows0, cols0, vals1, rows1, cols1, vals2, rows2, cols2, bias):
    T2 = _densify_t(vals2, rows2, cols2)
    T1 = _densify_t(vals1, rows1, cols1)
    T0 = _densify_t(vals0, rows0, cols0)
    T_stack = jnp.stack([T2, T1, T0])
    return _matmul_chain(U, T_stack, bias)
